# baseline (speedup 1.0000x reference)
import jax
import jax.numpy as jnp
from jax.experimental import pallas as pl
from jax.experimental.pallas import tpu as pltpu

_B = 32
_N = 64
_E = _N * _N
_H = 72
_S = 7
_L = 6
_NC = 2
_CW = 0.001
_ETOT = _B * _N * (_N - 1)


def _psi(t):
    return jnp.sign(t) * jnp.log(jnp.abs(t) + 1.0)


def _lorentz_body(scal_ref, x_ref,
                  emb_W, emb_b,
                  phi_e_W1, phi_e_g1, phi_e_be1, phi_e_W2, phi_e_b2,
                  phi_h_W1, phi_h_b1, phi_h_g, phi_h_be, phi_h_W2, phi_h_b2,
                  phi_x_W1, phi_x_b1, phi_x_W2, phi_m_W, phi_m_b,
                  dec_W1, dec_b1, dec_W2, dec_b2,
                  out_ref, haggT_scr, xaggT_scr, aT_scr, cT_scr,
                  nm_scr, dm_scr, xT_scr, rhs_scr):
    f32 = jnp.float32
    scal2 = scal_ref[:].reshape(_B * _N, _S)
    xv = x_ref[:]

    ei = jax.lax.broadcasted_iota(jnp.int32, (_N, _E), 1) // _N
    ej = jax.lax.broadcasted_iota(jnp.int32, (_N, _E), 1) % _N
    row = jax.lax.broadcasted_iota(jnp.int32, (_N, _E), 0)
    sel_i = (ei == row).astype(f32)
    sel_j = (ej == row).astype(f32)
    sel_d = sel_i - sel_j
    eTi = jax.lax.broadcasted_iota(jnp.int32, (_E, _N), 0) // _N
    eTj = jax.lax.broadcasted_iota(jnp.int32, (_E, _N), 0) % _N
    colT = jax.lax.broadcasted_iota(jnp.int32, (_E, _N), 1)
    selIm = ((eTi == colT) & (eTi != eTj)).astype(f32)
    selI = (eTi == colT).astype(f32)
    rhs_scr[0:_N] = sel_i
    rhs_scr[_N:2 * _N] = sel_j
    rhs_scr[2 * _N + 2:] = jnp.zeros((6, _E), f32)

    gvec = jnp.where(
        jax.lax.broadcasted_iota(jnp.int32, (1, 1, 4), 2) == 0, 1.0, -1.0
    ).astype(f32)
    mask2 = (jax.lax.broadcasted_iota(jnp.int32, (_N, _N), 0)
             != jax.lax.broadcasted_iota(jnp.int32, (_N, _N), 1)).astype(f32)

    h2 = (scal2 @ emb_W[:]) + emb_b[:]

    for l in range(_L):
        W1 = phi_e_W1[l]
        af = (h2 @ W1[0:_H])
        cf = (h2 @ W1[_H:2 * _H])
        a3 = af.reshape(_B, _N, _H)
        c3 = cf.reshape(_B, _N, _H)
        wn2 = W1[2 * _H:2 * _H + 1]
        wd2 = W1[2 * _H + 1:2 * _H + 2]

        xT = jnp.transpose(xv, (0, 2, 1))
        qd = jnp.sum(gvec * xv * xv, axis=-1, keepdims=True)
        qdT = jnp.sum(gvec.reshape(1, 4, 1) * xT * xT, axis=1,
                      keepdims=True)
        Qc = (xv[:, :, 0:1] * xT[:, 0:1, :] - xv[:, :, 1:2] * xT[:, 1:2, :]
              - xv[:, :, 2:3] * xT[:, 2:3, :] - xv[:, :, 3:4] * xT[:, 3:4, :])
        nmc = _psi(qd + qdT - 2.0 * Qc)
        dmc = _psi(Qc) * mask2

        Sn = jnp.sum(nmc)
        Sd = jnp.sum(dmc)
        Sn2 = jnp.sum(nmc * nmc)
        Sd2 = jnp.sum(dmc * dmc)
        Snd = jnp.sum(nmc * dmc)
        rn = jnp.sum(nmc, axis=2).reshape(_B * _N, 1)
        rd = jnp.sum(dmc, axis=2).reshape(_B * _N, 1)
        amean = jnp.sum(af, axis=0, keepdims=True) / (_B * _N)
        cmean = jnp.sum(cf, axis=0, keepdims=True) / (_B * _N)
        a0 = af - amean
        c0 = cf - cmean
        a03 = a0.reshape(_B, _N, _H)
        c03 = c0.reshape(_B, _N, _H)
        t_a0 = jnp.sum(a0, axis=0, keepdims=True)
        t_c0 = jnp.sum(c0, axis=0, keepdims=True)
        q_a0 = jnp.sum(a0 * a0, axis=0, keepdims=True)
        q_c0 = jnp.sum(c0 * c0, axis=0, keepdims=True)
        t_acd0 = jnp.sum(a0 * c0, axis=0, keepdims=True)
        Ab0 = jnp.sum(a03, axis=1)
        Cb0 = jnp.sum(c03, axis=1)
        t_cross0 = jnp.sum(Ab0 * Cb0, axis=0, keepdims=True)
        a_rn = jnp.sum(a0 * rn, axis=0, keepdims=True)
        c_rn = jnp.sum(c0 * rn, axis=0, keepdims=True)
        a_rd = jnp.sum(a0 * rd, axis=0, keepdims=True)
        c_rd = jnp.sum(c0 * rd, axis=0, keepdims=True)
        s1c = (_N - 1.0) * (t_a0 + t_c0) + Sn * wn2 + Sd * wd2
        s2c = ((_N - 1.0) * (q_a0 + q_c0) + 2.0 * (t_cross0 - t_acd0)
               + 2.0 * wn2 * (a_rn + c_rn) + 2.0 * wd2 * (a_rd + c_rd)
               + wn2 * wn2 * Sn2 + wd2 * wd2 * Sd2 + 2.0 * wn2 * wd2 * Snd)
        muc = s1c / _ETOT
        mu = muc + amean + cmean
        var = s2c / _ETOT - muc * muc
        sc = phi_e_g1[l:l + 1] * jax.lax.rsqrt(var + 1e-5)
        sh = phi_e_be1[l:l + 1] - mu * sc

        a3n = a3 * sc.reshape(1, 1, _H) + sh.reshape(1, 1, _H)
        c3n = c3 * sc.reshape(1, 1, _H)
        aT_scr[:] = jnp.transpose(a3n, (0, 2, 1))
        cT_scr[:] = jnp.transpose(c3n, (0, 2, 1))
        nm_scr[:] = nmc
        dm_scr[:] = dmc
        xT_scr[:] = xT
        Wnd = jnp.concatenate(
            [jnp.transpose(wn2 * sc), jnp.transpose(wd2 * sc)], axis=1)

        W2T = jnp.transpose(phi_e_W2[l])
        b2T = jnp.transpose(phi_e_b2[l:l + 1])
        wmC = phi_m_W[l]
        bm = phi_m_b[l]
        Wx1T = jnp.transpose(phi_x_W1[l])
        bx1T = jnp.transpose(phi_x_b1[l:l + 1])
        wx2C = phi_x_W2[l]

        lhs_pad = jnp.zeros((_H, 2 * _N + 8 - 2), f32)
        lhs_tail = jnp.concatenate([Wnd, lhs_pad[:, 0:6]], axis=1)

        def _pb(b, carry):
            aTb = aT_scr[b]
            cTb = cT_scr[b]
            rhs_scr[2 * _N:2 * _N + 1] = nm_scr[b].reshape(1, _E)
            rhs_scr[2 * _N + 1:2 * _N + 2] = dm_scr[b].reshape(1, _E)
            lhs = jnp.concatenate([aTb, cTb, lhs_tail], axis=1)
            rT = jnp.maximum((lhs @ rhs_scr[:]), 0.0)
            m2T = jnp.maximum((W2T @ rT) + b2T, 0.0)
            wgtT = jax.nn.sigmoid(
                jnp.sum(m2T * wmC, axis=0, keepdims=True) + bm)
            mT = m2T * wgtT
            haggT_scr[b] = (mT @ selIm)
            if l < _L - 1:
                t1T = jnp.maximum((Wx1T @ mT) + bx1T, 0.0)
                txT = jnp.sum(t1T * wx2C, axis=0, keepdims=True)
                xTb = xT_scr[b]
                xdT = (xTb @ sel_d)
                transT = jnp.clip(xdT * txT, -100.0, 100.0)
                xaggT_scr[b] = (transT @ selI)
            return carry

        jax.lax.fori_loop(0, _B, _pb, 0)

        if l < _L - 1:
            xv = xv + jnp.transpose(xaggT_scr[:], (0, 2, 1)) * (
                _CW / float(_N - 1))

        Wh1 = phi_h_W1[l]
        hagg2 = jnp.transpose(haggT_scr[:], (0, 2, 1)).reshape(_B * _N, _H)
        hh = ((h2 @ Wh1[0:_H]) + (hagg2 @ Wh1[_H:2 * _H])
              + (scal2 @ Wh1[2 * _H:2 * _H + _S]) + phi_h_b1[l:l + 1])
        hmu = jnp.mean(hh, axis=0, keepdims=True)
        hd = hh - hmu
        hvar = jnp.mean(hd * hd, axis=0, keepdims=True)
        hh = ((hh - hmu) * jax.lax.rsqrt(hvar + 1e-5)
              * phi_h_g[l:l + 1] + phi_h_be[l:l + 1])
        hh = jnp.maximum(hh, 0.0)
        h2 = h2 + (hh @ phi_h_W2[l]) + phi_h_b2[l:l + 1]

    hm = jnp.mean(h2.reshape(_B, _N, _H), axis=1)
    d1 = jnp.maximum((hm @ dec_W1[:]) + dec_b1[:], 0.0)
    out_ref[:] = (d1 @ dec_W2[:]) + dec_b2[:]


def kernel(scalars, x, params):
    p = params
    flat = (
        p['emb_W'], p['emb_b'].reshape(1, _H),
        p['phi_e_W1'], p['phi_e_g1'], p['phi_e_be1'], p['phi_e_W2'],
        p['phi_e_b2'],
        p['phi_h_W1'], p['phi_h_b1'], p['phi_h_g'], p['phi_h_be'],
        p['phi_h_W2'], p['phi_h_b2'],
        p['phi_x_W1'], p['phi_x_b1'], p['phi_x_W2'],
        p['phi_m_W'], p['phi_m_b'],
        p['dec_W1'], p['dec_b1'].reshape(1, _H),
        p['dec_W2'], p['dec_b2'].reshape(1, _NC),
    )
    return pl.pallas_call(
        _lorentz_body,
        out_shape=jax.ShapeDtypeStruct((_B, _NC), jnp.float32),
        scratch_shapes=[
            pltpu.VMEM((_B, _H, _N), jnp.float32),
            pltpu.VMEM((_B, 4, _N), jnp.float32),
            pltpu.VMEM((_B, _H, _N), jnp.float32),
            pltpu.VMEM((_B, _H, _N), jnp.float32),
            pltpu.VMEM((_B, _N, _N), jnp.float32),
            pltpu.VMEM((_B, _N, _N), jnp.float32),
            pltpu.VMEM((_B, 4, _N), jnp.float32),
            pltpu.VMEM((2 * _N + 8, _E), jnp.float32),
        ],
    )(scalars, x, *flat)


# speedup vs baseline: 1.0264x; 1.0264x over previous
import jax
import jax.numpy as jnp
from jax.experimental import pallas as pl
from jax.experimental.pallas import tpu as pltpu

_B = 32
_N = 64
_E = _N * _N
_H = 72
_S = 7
_L = 6
_NC = 2
_CW = 0.001
_ETOT = _B * _N * (_N - 1)


def _psi(t):
    return jnp.sign(t) * jnp.log(jnp.abs(t) + 1.0)


def _lorentz_body(scal_ref, x_ref,
                  emb_W, emb_b,
                  phi_e_W1, phi_e_g1, phi_e_be1, phi_e_W2, phi_e_b2,
                  phi_h_W1, phi_h_b1, phi_h_g, phi_h_be, phi_h_W2, phi_h_b2,
                  phi_x_W1, phi_x_b1, phi_x_W2, phi_m_W, phi_m_b,
                  dec_W1, dec_b1, dec_W2, dec_b2,
                  sel_i_ref, sel_j_ref, sel_d_ref, selIm_ref,
                  out_ref, haggT_scr, xaggT_scr, aT_scr, cT_scr,
                  nm_scr, dm_scr, xT_scr, rhs_scr):
    f32 = jnp.float32
    scal2 = scal_ref[:].reshape(_B * _N, _S)
    xv = x_ref[:]

    sel_d = sel_d_ref[:]
    selIm = selIm_ref[:]
    rhs_scr[0:_N] = sel_i_ref[:]
    rhs_scr[_N:2 * _N] = sel_j_ref[:]
    rhs_scr[2 * _N + 2:] = jnp.zeros((6, _E), f32)

    gvec = jnp.where(
        jax.lax.broadcasted_iota(jnp.int32, (1, 1, 4), 2) == 0, 1.0, -1.0
    ).astype(f32)
    mask2 = (jax.lax.broadcasted_iota(jnp.int32, (_N, _N), 0)
             != jax.lax.broadcasted_iota(jnp.int32, (_N, _N), 1)).astype(f32)

    h2 = (scal2 @ emb_W[:]) + emb_b[:]

    for l in range(_L):
        W1 = phi_e_W1[l]
        af = (h2 @ W1[0:_H])
        cf = (h2 @ W1[_H:2 * _H])
        a3 = af.reshape(_B, _N, _H)
        c3 = cf.reshape(_B, _N, _H)
        wn2 = W1[2 * _H:2 * _H + 1]
        wd2 = W1[2 * _H + 1:2 * _H + 2]

        xT = jnp.transpose(xv, (0, 2, 1))
        qd = jnp.sum(gvec * xv * xv, axis=-1, keepdims=True)
        qdT = jnp.sum(gvec.reshape(1, 4, 1) * xT * xT, axis=1,
                      keepdims=True)
        Qc = (xv[:, :, 0:1] * xT[:, 0:1, :] - xv[:, :, 1:2] * xT[:, 1:2, :]
              - xv[:, :, 2:3] * xT[:, 2:3, :] - xv[:, :, 3:4] * xT[:, 3:4, :])
        nmc = _psi(qd + qdT - 2.0 * Qc)
        dmc = _psi(Qc) * mask2

        Sn = jnp.sum(nmc)
        Sd = jnp.sum(dmc)
        Sn2 = jnp.sum(nmc * nmc)
        Sd2 = jnp.sum(dmc * dmc)
        Snd = jnp.sum(nmc * dmc)
        rn = jnp.sum(nmc, axis=2).reshape(_B * _N, 1)
        rd = jnp.sum(dmc, axis=2).reshape(_B * _N, 1)
        amean = jnp.sum(af, axis=0, keepdims=True) / (_B * _N)
        cmean = jnp.sum(cf, axis=0, keepdims=True) / (_B * _N)
        a0 = af - amean
        c0 = cf - cmean
        a03 = a0.reshape(_B, _N, _H)
        c03 = c0.reshape(_B, _N, _H)
        t_a0 = jnp.sum(a0, axis=0, keepdims=True)
        t_c0 = jnp.sum(c0, axis=0, keepdims=True)
        q_a0 = jnp.sum(a0 * a0, axis=0, keepdims=True)
        q_c0 = jnp.sum(c0 * c0, axis=0, keepdims=True)
        t_acd0 = jnp.sum(a0 * c0, axis=0, keepdims=True)
        Ab0 = jnp.sum(a03, axis=1)
        Cb0 = jnp.sum(c03, axis=1)
        t_cross0 = jnp.sum(Ab0 * Cb0, axis=0, keepdims=True)
        a_rn = jnp.sum(a0 * rn, axis=0, keepdims=True)
        c_rn = jnp.sum(c0 * rn, axis=0, keepdims=True)
        a_rd = jnp.sum(a0 * rd, axis=0, keepdims=True)
        c_rd = jnp.sum(c0 * rd, axis=0, keepdims=True)
        s1c = (_N - 1.0) * (t_a0 + t_c0) + Sn * wn2 + Sd * wd2
        s2c = ((_N - 1.0) * (q_a0 + q_c0) + 2.0 * (t_cross0 - t_acd0)
               + 2.0 * wn2 * (a_rn + c_rn) + 2.0 * wd2 * (a_rd + c_rd)
               + wn2 * wn2 * Sn2 + wd2 * wd2 * Sd2 + 2.0 * wn2 * wd2 * Snd)
        muc = s1c / _ETOT
        mu = muc + amean + cmean
        var = s2c / _ETOT - muc * muc
        sc = phi_e_g1[l:l + 1] * jax.lax.rsqrt(var + 1e-5)
        sh = phi_e_be1[l:l + 1] - mu * sc

        a3n = a3 * sc.reshape(1, 1, _H) + sh.reshape(1, 1, _H)
        c3n = c3 * sc.reshape(1, 1, _H)
        aT_scr[:] = jnp.transpose(a3n, (0, 2, 1))
        cT_scr[:] = jnp.transpose(c3n, (0, 2, 1))
        nm_scr[:] = nmc
        dm_scr[:] = dmc
        xT_scr[:] = xT
        Wnd = jnp.concatenate(
            [jnp.transpose(wn2 * sc), jnp.transpose(wd2 * sc)], axis=1)

        W2T = jnp.transpose(phi_e_W2[l])
        b2T = jnp.transpose(phi_e_b2[l:l + 1])
        wmC = phi_m_W[l]
        bm = phi_m_b[l]
        Wx1T = jnp.transpose(phi_x_W1[l])
        bx1T = jnp.transpose(phi_x_b1[l:l + 1])
        wx2C = phi_x_W2[l]

        lhs_pad = jnp.zeros((_H, 2 * _N + 8 - 2), f32)
        lhs_tail = jnp.concatenate([Wnd, lhs_pad[:, 0:6]], axis=1)

        def _pb(b, carry):
            aTb = aT_scr[b]
            cTb = cT_scr[b]
            rhs_scr[2 * _N:2 * _N + 1] = nm_scr[b].reshape(1, _E)
            rhs_scr[2 * _N + 1:2 * _N + 2] = dm_scr[b].reshape(1, _E)
            lhs = jnp.concatenate([aTb, cTb, lhs_tail], axis=1)
            rT = jnp.maximum((lhs @ rhs_scr[:]), 0.0)
            m2T = jnp.maximum((W2T @ rT) + b2T, 0.0)
            wgtT = jax.nn.sigmoid(
                jnp.sum(m2T * wmC, axis=0, keepdims=True) + bm)
            mT = m2T * wgtT
            haggT_scr[b] = (mT @ selIm)
            if l < _L - 1:
                t1T = jnp.maximum((Wx1T @ mT) + bx1T, 0.0)
                txT = jnp.sum(t1T * wx2C, axis=0, keepdims=True)
                xTb = xT_scr[b]
                xdT = (xTb @ sel_d)
                transT = jnp.clip(xdT * txT, -100.0, 100.0)
                xaggT_scr[b] = (transT @ selIm)
            return carry

        jax.lax.fori_loop(0, _B, _pb, 0)

        if l < _L - 1:
            xv = xv + jnp.transpose(xaggT_scr[:], (0, 2, 1)) * (
                _CW / float(_N - 1))

        Wh1 = phi_h_W1[l]
        hagg2 = jnp.transpose(haggT_scr[:], (0, 2, 1)).reshape(_B * _N, _H)
        hh = ((h2 @ Wh1[0:_H]) + (hagg2 @ Wh1[_H:2 * _H])
              + (scal2 @ Wh1[2 * _H:2 * _H + _S]) + phi_h_b1[l:l + 1])
        hmu = jnp.mean(hh, axis=0, keepdims=True)
        hd = hh - hmu
        hvar = jnp.mean(hd * hd, axis=0, keepdims=True)
        hh = ((hh - hmu) * jax.lax.rsqrt(hvar + 1e-5)
              * phi_h_g[l:l + 1] + phi_h_be[l:l + 1])
        hh = jnp.maximum(hh, 0.0)
        h2 = h2 + (hh @ phi_h_W2[l]) + phi_h_b2[l:l + 1]

    hm = jnp.mean(h2.reshape(_B, _N, _H), axis=1)
    d1 = jnp.maximum((hm @ dec_W1[:]) + dec_b1[:], 0.0)
    out_ref[:] = (d1 @ dec_W2[:]) + dec_b2[:]


import numpy as _np


def _sel_i():
    e = _np.arange(_E)
    return jnp.asarray((e // _N == _np.arange(_N)[:, None]).astype(_np.float32))


def _sel_j():
    e = _np.arange(_E)
    return jnp.asarray((e % _N == _np.arange(_N)[:, None]).astype(_np.float32))


def _sel_d():
    return _sel_i() - _sel_j()


def _selIm():
    e = _np.arange(_E)[:, None]
    m = (e // _N == _np.arange(_N)[None, :]) & (e // _N != e % _N)
    return jnp.asarray(m.astype(_np.float32))


def kernel(scalars, x, params):
    p = params
    flat = (
        p['emb_W'], p['emb_b'].reshape(1, _H),
        p['phi_e_W1'], p['phi_e_g1'], p['phi_e_be1'], p['phi_e_W2'],
        p['phi_e_b2'],
        p['phi_h_W1'], p['phi_h_b1'], p['phi_h_g'], p['phi_h_be'],
        p['phi_h_W2'], p['phi_h_b2'],
        p['phi_x_W1'], p['phi_x_b1'], p['phi_x_W2'],
        p['phi_m_W'], p['phi_m_b'],
        p['dec_W1'], p['dec_b1'].reshape(1, _H),
        p['dec_W2'], p['dec_b2'].reshape(1, _NC),
        _sel_i(), _sel_j(), _sel_d(), _selIm(),
    )
    return pl.pallas_call(
        _lorentz_body,
        out_shape=jax.ShapeDtypeStruct((_B, _NC), jnp.float32),
        scratch_shapes=[
            pltpu.VMEM((_B, _H, _N), jnp.float32),
            pltpu.VMEM((_B, 4, _N), jnp.float32),
            pltpu.VMEM((_B, _H, _N), jnp.float32),
            pltpu.VMEM((_B, _H, _N), jnp.float32),
            pltpu.VMEM((_B, _N, _N), jnp.float32),
            pltpu.VMEM((_B, _N, _N), jnp.float32),
            pltpu.VMEM((_B, 4, _N), jnp.float32),
            pltpu.VMEM((2 * _N + 8, _E), jnp.float32),
        ],
    )(scalars, x, *flat)
